# baseline (speedup 1.0000x reference)
import jax
import jax.numpy as jnp
from jax.experimental import pallas as pl
from jax.experimental.pallas import tpu as pltpu

_G = 512


def _pool_body(batch_ref, x_ref, Wp_ref, bp_ref, Ws_ref, bs_ref,
               out_ref, denom_ref):
    i = pl.program_id(0)
    k = pl.num_programs(0)
    x = x_ref[...]
    h = jnp.tanh(
        jnp.dot(x, Wp_ref[...], preferred_element_type=jnp.float32)
        + bp_ref[...])
    s = jnp.dot(h, Ws_ref[...], preferred_element_type=jnp.float32) \
        + bs_ref[0, 0]
    e = jnp.exp(s)

    bids = batch_ref[0]
    gi = jax.lax.broadcasted_iota(jnp.int32, (_G, bids.shape[1]), 0)
    onehot = (gi == bids).astype(jnp.bfloat16)
    contrib = jnp.dot(onehot, (x * e).astype(jnp.bfloat16),
                      preferred_element_type=jnp.float32)
    dcontrib = jnp.dot(onehot, e.astype(jnp.bfloat16),
                       preferred_element_type=jnp.float32)

    @pl.when(i == 0)
    def _init():
        out_ref[...] = contrib
        denom_ref[...] = dcontrib

    @pl.when(i > 0)
    def _acc():
        out_ref[...] += contrib
        denom_ref[...] += dcontrib

    @pl.when(i == k - 1)
    def _normalize():
        den = denom_ref[...]
        den = jnp.where(den == 0.0, 1.0, den)
        out_ref[...] = out_ref[...] / den


def kernel(x, batch, Wp, bp, Ws, bs):
    n, d = x.shape
    blk = 1000
    k = n // blk
    assert k * blk == n
    batch3 = batch.reshape(k, 1, blk)
    bp2 = bp.reshape(1, d)
    ws2 = Ws.reshape(d, 1)
    bs2 = bs.reshape(1, 1)
    return pl.pallas_call(
        _pool_body,
        grid=(k,),
        in_specs=[
            pl.BlockSpec((1, 1, blk), lambda i: (i, 0, 0)),
            pl.BlockSpec((blk, d), lambda i: (i, 0)),
            pl.BlockSpec((d, d), lambda i: (0, 0)),
            pl.BlockSpec((1, d), lambda i: (0, 0)),
            pl.BlockSpec((d, 1), lambda i: (0, 0)),
            pl.BlockSpec((1, 1), lambda i: (0, 0)),
        ],
        out_specs=pl.BlockSpec((_G, d), lambda i: (0, 0)),
        out_shape=jax.ShapeDtypeStruct((_G, d), jnp.float32),
        scratch_shapes=[pltpu.VMEM((_G, 1), jnp.float32)],
    )(batch3, x, Wp, bp2, ws2, bs2)


# speedup vs baseline: 1.0077x; 1.0077x over previous
import jax
import jax.numpy as jnp
from jax.experimental import pallas as pl
from jax.experimental.pallas import tpu as pltpu

_G = 512


def _pool_body(batch_ref, x_ref, Wp_ref, bp_ref, Ws_ref, bs_ref,
               out_ref, denom_ref):
    i = pl.program_id(0)
    k = pl.num_programs(0)
    x = x_ref[...]
    h = jnp.tanh(
        jnp.dot(x.astype(jnp.bfloat16), Wp_ref[...].astype(jnp.bfloat16),
                preferred_element_type=jnp.float32)
        + bp_ref[...])
    s = jnp.dot(h, Ws_ref[...], preferred_element_type=jnp.float32) \
        + bs_ref[0, 0]
    e = jnp.exp(s)

    bids = batch_ref[0]
    gi = jax.lax.broadcasted_iota(jnp.int32, (_G, bids.shape[1]), 0)
    onehot = (gi == bids).astype(jnp.bfloat16)
    contrib = jnp.dot(onehot, (x * e).astype(jnp.bfloat16),
                      preferred_element_type=jnp.float32)
    dcontrib = jnp.dot(onehot, e.astype(jnp.bfloat16),
                       preferred_element_type=jnp.float32)

    @pl.when(i == 0)
    def _init():
        out_ref[...] = contrib
        denom_ref[...] = dcontrib

    @pl.when(i > 0)
    def _acc():
        out_ref[...] += contrib
        denom_ref[...] += dcontrib

    @pl.when(i == k - 1)
    def _normalize():
        den = denom_ref[...]
        den = jnp.where(den == 0.0, 1.0, den)
        out_ref[...] = out_ref[...] / den


def kernel(x, batch, Wp, bp, Ws, bs):
    n, d = x.shape
    blk = 1000
    k = n // blk
    assert k * blk == n
    batch3 = batch.reshape(k, 1, blk)
    bp2 = bp.reshape(1, d)
    ws2 = Ws.reshape(d, 1)
    bs2 = bs.reshape(1, 1)
    return pl.pallas_call(
        _pool_body,
        grid=(k,),
        in_specs=[
            pl.BlockSpec((1, 1, blk), lambda i: (i, 0, 0)),
            pl.BlockSpec((blk, d), lambda i: (i, 0)),
            pl.BlockSpec((d, d), lambda i: (0, 0)),
            pl.BlockSpec((1, d), lambda i: (0, 0)),
            pl.BlockSpec((d, 1), lambda i: (0, 0)),
            pl.BlockSpec((1, 1), lambda i: (0, 0)),
        ],
        out_specs=pl.BlockSpec((_G, d), lambda i: (0, 0)),
        out_shape=jax.ShapeDtypeStruct((_G, d), jnp.float32),
        scratch_shapes=[pltpu.VMEM((_G, 1), jnp.float32)],
    )(batch3, x, Wp, bp2, ws2, bs2)


# speedup vs baseline: 1.1896x; 1.1805x over previous
import jax
import jax.numpy as jnp
from jax.experimental import pallas as pl
from jax.experimental.pallas import tpu as pltpu

_G = 512


def _pool_body(batch_ref, x_ref, Wp_ref, bp_ref, Ws_ref, bs_ref,
               out_ref, denom_ref):
    i = pl.program_id(0)
    k = pl.num_programs(0)
    x = x_ref[...]
    h = jnp.tanh(
        jnp.dot(x.astype(jnp.bfloat16), Wp_ref[...].astype(jnp.bfloat16),
                preferred_element_type=jnp.float32)
        + bp_ref[...])
    s = jnp.dot(h, Ws_ref[...], preferred_element_type=jnp.float32) \
        + bs_ref[0, 0]
    e = jnp.exp(s)

    bids = batch_ref[0]
    gi = jax.lax.broadcasted_iota(jnp.int32, (_G, bids.shape[1]), 0)
    onehot = (gi == bids).astype(jnp.bfloat16)
    contrib = jnp.dot(onehot, (x * e).astype(jnp.bfloat16),
                      preferred_element_type=jnp.float32)
    dcontrib = jnp.dot(onehot, e.astype(jnp.bfloat16),
                       preferred_element_type=jnp.float32)

    @pl.when(i == 0)
    def _init():
        out_ref[...] = contrib
        denom_ref[...] = dcontrib

    @pl.when(i > 0)
    def _acc():
        out_ref[...] += contrib
        denom_ref[...] += dcontrib

    @pl.when(i == k - 1)
    def _normalize():
        den = denom_ref[...]
        den = jnp.where(den == 0.0, 1.0, den)
        out_ref[...] = out_ref[...] / den


def kernel(x, batch, Wp, bp, Ws, bs):
    n, d = x.shape
    blk = 2000
    k = n // blk
    assert k * blk == n
    batch3 = batch.reshape(k, 1, blk)
    bp2 = bp.reshape(1, d)
    ws2 = Ws.reshape(d, 1)
    bs2 = bs.reshape(1, 1)
    return pl.pallas_call(
        _pool_body,
        grid=(k,),
        in_specs=[
            pl.BlockSpec((1, 1, blk), lambda i: (i, 0, 0)),
            pl.BlockSpec((blk, d), lambda i: (i, 0)),
            pl.BlockSpec((d, d), lambda i: (0, 0)),
            pl.BlockSpec((1, d), lambda i: (0, 0)),
            pl.BlockSpec((d, 1), lambda i: (0, 0)),
            pl.BlockSpec((1, 1), lambda i: (0, 0)),
        ],
        out_specs=pl.BlockSpec((_G, d), lambda i: (0, 0)),
        out_shape=jax.ShapeDtypeStruct((_G, d), jnp.float32),
        scratch_shapes=[pltpu.VMEM((_G, 1), jnp.float32)],
    )(batch3, x, Wp, bp2, ws2, bs2)


# speedup vs baseline: 1.3145x; 1.1051x over previous
import jax
import jax.numpy as jnp
from jax.experimental import pallas as pl
from jax.experimental.pallas import tpu as pltpu

_G = 512


def _pool_body(batch_ref, x_ref, Wp_ref, bp_ref, Ws_ref, bs_ref,
               out_ref, denom_ref):
    i = pl.program_id(0)
    k = pl.num_programs(0)
    x = x_ref[...]
    h = jnp.tanh(
        jnp.dot(x.astype(jnp.bfloat16), Wp_ref[...].astype(jnp.bfloat16),
                preferred_element_type=jnp.float32)
        + bp_ref[...])
    s = jnp.dot(h, Ws_ref[...], preferred_element_type=jnp.float32) \
        + bs_ref[0, 0]
    e = jnp.exp(s)

    bids = batch_ref[0]
    gi = jax.lax.broadcasted_iota(jnp.int32, (_G, bids.shape[1]), 0)
    onehot = (gi == bids).astype(jnp.bfloat16)
    contrib = jnp.dot(onehot, (x * e).astype(jnp.bfloat16),
                      preferred_element_type=jnp.float32)
    dcontrib = jnp.dot(onehot, e.astype(jnp.bfloat16),
                       preferred_element_type=jnp.float32)

    @pl.when(i == 0)
    def _init():
        out_ref[...] = contrib
        denom_ref[...] = dcontrib

    @pl.when(i > 0)
    def _acc():
        out_ref[...] += contrib
        denom_ref[...] += dcontrib

    @pl.when(i == k - 1)
    def _normalize():
        den = denom_ref[...]
        den = jnp.where(den == 0.0, 1.0, den)
        out_ref[...] = out_ref[...] / den


def kernel(x, batch, Wp, bp, Ws, bs):
    n, d = x.shape
    blk = 5000
    k = n // blk
    assert k * blk == n
    batch3 = batch.reshape(k, 1, blk)
    bp2 = bp.reshape(1, d)
    ws2 = Ws.reshape(d, 1)
    bs2 = bs.reshape(1, 1)
    return pl.pallas_call(
        _pool_body,
        grid=(k,),
        in_specs=[
            pl.BlockSpec((1, 1, blk), lambda i: (i, 0, 0)),
            pl.BlockSpec((blk, d), lambda i: (i, 0)),
            pl.BlockSpec((d, d), lambda i: (0, 0)),
            pl.BlockSpec((1, d), lambda i: (0, 0)),
            pl.BlockSpec((d, 1), lambda i: (0, 0)),
            pl.BlockSpec((1, 1), lambda i: (0, 0)),
        ],
        out_specs=pl.BlockSpec((_G, d), lambda i: (0, 0)),
        out_shape=jax.ShapeDtypeStruct((_G, d), jnp.float32),
        scratch_shapes=[pltpu.VMEM((_G, 1), jnp.float32)],
    )(batch3, x, Wp, bp2, ws2, bs2)


# speedup vs baseline: 1.3321x; 1.0134x over previous
import jax
import jax.numpy as jnp
from jax.experimental import pallas as pl
from jax.experimental.pallas import tpu as pltpu

_G = 512


def _pool_body(batch_ref, x_ref, Wp_ref, bp_ref, Ws_ref, bs_ref,
               out_ref, denom_ref):
    i = pl.program_id(0)
    k = pl.num_programs(0)
    x = x_ref[...]
    h = jnp.tanh(
        jnp.dot(x.astype(jnp.bfloat16), Wp_ref[...].astype(jnp.bfloat16),
                preferred_element_type=jnp.float32)
        + bp_ref[...])
    s = jnp.dot(h, Ws_ref[...], preferred_element_type=jnp.float32) \
        + bs_ref[0, 0]
    e = jnp.exp(s)

    bids = batch_ref[0]
    gi = jax.lax.broadcasted_iota(jnp.int32, (_G, bids.shape[1]), 0)
    onehot = (gi == bids).astype(jnp.bfloat16)
    contrib = jnp.dot(onehot, (x * e).astype(jnp.bfloat16),
                      preferred_element_type=jnp.float32)
    dcontrib = jnp.dot(onehot, e.astype(jnp.bfloat16),
                       preferred_element_type=jnp.float32)

    @pl.when(i == 0)
    def _init():
        out_ref[...] = contrib
        denom_ref[...] = dcontrib

    @pl.when(i > 0)
    def _acc():
        out_ref[...] += contrib
        denom_ref[...] += dcontrib

    @pl.when(i == k - 1)
    def _normalize():
        den = denom_ref[...]
        den = jnp.where(den == 0.0, 1.0, den)
        out_ref[...] = out_ref[...] / den


def kernel(x, batch, Wp, bp, Ws, bs):
    n, d = x.shape
    blk = 10000
    k = n // blk
    assert k * blk == n
    batch3 = batch.reshape(k, 1, blk)
    bp2 = bp.reshape(1, d)
    ws2 = Ws.reshape(d, 1)
    bs2 = bs.reshape(1, 1)
    return pl.pallas_call(
        _pool_body,
        grid=(k,),
        in_specs=[
            pl.BlockSpec((1, 1, blk), lambda i: (i, 0, 0)),
            pl.BlockSpec((blk, d), lambda i: (i, 0)),
            pl.BlockSpec((d, d), lambda i: (0, 0)),
            pl.BlockSpec((1, d), lambda i: (0, 0)),
            pl.BlockSpec((d, 1), lambda i: (0, 0)),
            pl.BlockSpec((1, 1), lambda i: (0, 0)),
        ],
        out_specs=pl.BlockSpec((_G, d), lambda i: (0, 0)),
        out_shape=jax.ShapeDtypeStruct((_G, d), jnp.float32),
        scratch_shapes=[pltpu.VMEM((_G, 1), jnp.float32)],
    )(batch3, x, Wp, bp2, ws2, bs2)
